# baseline (speedup 1.0000x reference)
import functools

import jax
import jax.numpy as jnp
from jax import lax
from jax.experimental import pallas as pl
from jax.experimental.pallas import tpu as pltpu
from jax.experimental.pallas import tpu_sc as plsc

BATCH = 16384
MF_DIM = 32
MLP_HALF = 16
MLP_DIM = 32
NC = 2
NS = 16
NW = NC * NS
B_PER_W = BATCH // NW
CHUNK = 64
N_CHUNK = B_PER_W // CHUNK


def _sc_gather_body(user, item, mfu_t, mfi_t, mlpu_t, mlpi_t,
                    out_mf, out_mlp,
                    idx_u, idx_i, g_mfu, g_mfi, g_mlu, g_mli,
                    gb_mfu, gb_mfi, gb_mlu, gb_mli, st_mf, st_mlp, sem):
    wid = lax.axis_index("s") * NC + lax.axis_index("c")
    base = wid * B_PER_W
    pltpu.sync_copy(user.at[pl.ds(base, B_PER_W)], idx_u)
    pltpu.sync_copy(item.at[pl.ds(base, B_PER_W)], idx_i)

    def chunk_body(c, _):
        off = c * CHUNK

        def mkidx(g, _):
            vu = idx_u[pl.ds(off + g * 16, 16)]
            vi = idx_i[pl.ds(off + g * 16, 16)]
            s = pl.ds(g * 16, 16)
            g_mfu[s] = lax.shift_right_logical(vu, 2)
            g_mfi[s] = lax.shift_right_logical(vi, 2)
            g_mlu[s] = lax.shift_right_logical(vu, 3)
            g_mli[s] = lax.shift_right_logical(vi, 3)
            return _

        lax.fori_loop(0, CHUNK // 16, mkidx, 0)

        pltpu.async_copy(mfu_t.at[g_mfu], gb_mfu, sem)
        pltpu.async_copy(mfi_t.at[g_mfi], gb_mfi, sem)
        pltpu.async_copy(mlpu_t.at[g_mlu], gb_mlu, sem)
        pltpu.async_copy(mlpi_t.at[g_mli], gb_mli, sem)
        pltpu.make_async_copy(mfu_t.at[pl.ds(0, CHUNK)], gb_mfu, sem).wait()
        pltpu.make_async_copy(mfi_t.at[pl.ds(0, CHUNK)], gb_mfi, sem).wait()
        pltpu.make_async_copy(mlpu_t.at[pl.ds(0, CHUNK)], gb_mlu, sem).wait()
        pltpu.make_async_copy(mlpi_t.at[pl.ds(0, CHUNK)], gb_mli, sem).wait()

        def extract(g, _):
            vu = idx_u[pl.ds(off + g * 16, 16)]
            vi = idx_i[pl.ds(off + g * 16, 16)]
            for k in range(16):
                s = g * 16 + k
                ou = pl.multiple_of((vu[k] & 3) * MF_DIM, MF_DIM)
                oi = pl.multiple_of((vi[k] & 3) * MF_DIM, MF_DIM)
                olu = pl.multiple_of((vu[k] & 7) * MLP_HALF, MLP_HALF)
                oli = pl.multiple_of((vi[k] & 7) * MLP_HALF, MLP_HALF)
                a0 = gb_mfu[s, pl.ds(ou, 16)]
                a1 = gb_mfu[s, pl.ds(ou + 16, 16)]
                b0 = gb_mfi[s, pl.ds(oi, 16)]
                b1 = gb_mfi[s, pl.ds(oi + 16, 16)]
                st_mf[s, pl.ds(0, 16)] = a0 * b0
                st_mf[s, pl.ds(16, 16)] = a1 * b1
                st_mlp[s, pl.ds(0, 16)] = gb_mlu[s, pl.ds(olu, 16)]
                st_mlp[s, pl.ds(16, 16)] = gb_mli[s, pl.ds(oli, 16)]
            return _

        lax.fori_loop(0, CHUNK // 16, extract, 0)
        row = pl.ds(base + off, CHUNK)
        pltpu.sync_copy(st_mf, out_mf.at[row])
        pltpu.sync_copy(st_mlp, out_mlp.at[row])
        return _

    lax.fori_loop(0, N_CHUNK, chunk_body, 0)


def _sc_gather(user, item, mfu_t, mfi_t, mlpu_t, mlpi_t):
    mesh = plsc.VectorSubcoreMesh(core_axis_name="c", subcore_axis_name="s")
    f32 = jnp.float32
    i32 = jnp.int32
    run = functools.partial(
        pl.kernel,
        mesh=mesh,
        out_type=[
            jax.ShapeDtypeStruct((BATCH, MF_DIM), f32),
            jax.ShapeDtypeStruct((BATCH, MLP_DIM), f32),
        ],
        scratch_types=[
            pltpu.VMEM((B_PER_W,), i32),
            pltpu.VMEM((B_PER_W,), i32),
            pltpu.VMEM((CHUNK,), i32),
            pltpu.VMEM((CHUNK,), i32),
            pltpu.VMEM((CHUNK,), i32),
            pltpu.VMEM((CHUNK,), i32),
            pltpu.VMEM((CHUNK, 128), f32),
            pltpu.VMEM((CHUNK, 128), f32),
            pltpu.VMEM((CHUNK, 128), f32),
            pltpu.VMEM((CHUNK, 128), f32),
            pltpu.VMEM((CHUNK, MF_DIM), f32),
            pltpu.VMEM((CHUNK, MLP_DIM), f32),
            pltpu.SemaphoreType.DMA,
        ],
    )(_sc_gather_body)
    return run(user, item, mfu_t, mfi_t, mlpu_t, mlpi_t)


def _tc_mlp_body(xmf, xmlp, w0, b0, w1, b1, w2, b2, w3, b3, wfm, wfp, bf, out):
    f32 = jnp.float32
    h = xmlp[...]
    for w, b in ((w0, b0), (w1, b1), (w2, b2), (w3, b3)):
        h = jnp.maximum(jnp.dot(h, w[...], preferred_element_type=f32) + b[...], 0.0)
    logit = (jnp.dot(xmf[...], wfm[...], preferred_element_type=f32)
             + jnp.dot(h, wfp[...], preferred_element_type=f32)
             + bf[...])
    out[...] = jax.nn.sigmoid(logit)


def _tc_mlp(xmf, xmlp, W0, b0, W1, b1, W2, b2, W3, b3, Wf, bf):
    R = 2048
    grid = (BATCH // R,)
    D = MLP_DIM
    rows = lambda d: pl.BlockSpec((R, d), lambda i: (i, 0))
    full = lambda a, b: pl.BlockSpec((a, b), lambda i: (0, 0))
    in_specs = [
        rows(D), rows(D),
        full(D, D), full(1, D),
        full(D, D), full(1, D),
        full(D, D), full(1, D),
        full(D, D), full(1, D),
        full(D, 1), full(D, 1), full(1, 1),
    ]
    out_spec = pl.BlockSpec((R, 1), lambda i: (i, 0))
    args = (
        xmf, xmlp,
        W0, b0.reshape(1, D),
        W1, b1.reshape(1, D),
        W2, b2.reshape(1, D),
        W3, b3.reshape(1, D),
        Wf[:D], Wf[D:], bf.reshape(1, 1),
    )
    return pl.pallas_call(
        _tc_mlp_body,
        grid=grid,
        in_specs=in_specs,
        out_specs=out_spec,
        out_shape=jax.ShapeDtypeStruct((BATCH, 1), jnp.float32),
    )(*args)


def kernel(user, item, mf_user_embed, mf_item_embed, mlp_user_embed,
           mlp_item_embed, W0, b0, W1, b1, W2, b2, W3, b3, Wf, bf):
    t_mfu = mf_user_embed.reshape(-1, 128)
    t_mfi = mf_item_embed.reshape(-1, 128)
    t_mlu = mlp_user_embed.reshape(-1, 128)
    t_mli = mlp_item_embed.reshape(-1, 128)
    xmf, xmlp = _sc_gather(user, item, t_mfu, t_mfi, t_mlu, t_mli)
    return _tc_mlp(xmf, xmlp, W0, b0, W1, b1, W2, b2, W3, b3, Wf, bf)


# speedup vs baseline: 4.2530x; 4.2530x over previous
import functools

import jax
import jax.numpy as jnp
from jax import lax
from jax.experimental import pallas as pl
from jax.experimental.pallas import tpu as pltpu
from jax.experimental.pallas import tpu_sc as plsc

BATCH = 16384
MF_DIM = 32
MLP_HALF = 16
MLP_DIM = 32
NC = 2
NS = 16
NW = NC * NS
B_PER_W = BATCH // NW
SUB = 8


def _sc_gather_body(user, item, mfu_t, mfi_t, mlpu_t, mlpi_t,
                    out_mf, out_mlp,
                    idx_u, idx_i, bmfu, bmfi, bmlu, bmli, st_mf, st_mlp, sem):
    wid = lax.axis_index("s") * NC + lax.axis_index("c")
    base = wid * B_PER_W
    pltpu.sync_copy(user.at[pl.ds(base, B_PER_W)], idx_u)
    pltpu.sync_copy(item.at[pl.ds(base, B_PER_W)], idx_i)
    lanes = lax.iota(jnp.int32, 16)

    def step(t, _):
        vu = idx_u[pl.ds(t * 16, 16)]
        vi = idx_i[pl.ds(t * 16, 16)]
        for half in range(2):
            for k in range(SUB):
                iu = vu[half * SUB + k]
                ii = vi[half * SUB + k]
                cu = pl.multiple_of((iu >> 7) * 128, 128)
                ci = pl.multiple_of((ii >> 7) * 128, 128)
                pltpu.async_copy(mfu_t.at[:, pl.ds(cu, 128)], bmfu.at[k], sem)
                pltpu.async_copy(mfi_t.at[:, pl.ds(ci, 128)], bmfi.at[k], sem)
                pltpu.async_copy(mlpu_t.at[:, pl.ds(cu, 128)], bmlu.at[k], sem)
                pltpu.async_copy(mlpi_t.at[:, pl.ds(ci, 128)], bmli.at[k], sem)
            for k in range(SUB):
                pltpu.make_async_copy(mfu_t.at[:, pl.ds(0, 128)], bmfu.at[k], sem).wait()
                pltpu.make_async_copy(mfi_t.at[:, pl.ds(0, 128)], bmfi.at[k], sem).wait()
                pltpu.make_async_copy(mlpu_t.at[:, pl.ds(0, 128)], bmlu.at[k], sem).wait()
                pltpu.make_async_copy(mlpi_t.at[:, pl.ds(0, 128)], bmli.at[k], sem).wait()
            for k in range(SUB):
                s = half * SUB + k
                lu = jnp.full((16,), vu[s] & 127, jnp.int32)
                li = jnp.full((16,), vi[s] & 127, jnp.int32)
                a0 = plsc.load_gather(bmfu.at[k], [lanes, lu])
                a1 = plsc.load_gather(bmfu.at[k], [lanes + 16, lu])
                b0 = plsc.load_gather(bmfi.at[k], [lanes, li])
                b1 = plsc.load_gather(bmfi.at[k], [lanes + 16, li])
                st_mf[s, pl.ds(0, 16)] = a0 * b0
                st_mf[s, pl.ds(16, 16)] = a1 * b1
                st_mlp[s, pl.ds(0, 16)] = plsc.load_gather(bmlu.at[k], [lanes, lu])
                st_mlp[s, pl.ds(16, 16)] = plsc.load_gather(bmli.at[k], [lanes, li])
        row = pl.ds(base + t * 16, 16)
        pltpu.sync_copy(st_mf, out_mf.at[row])
        pltpu.sync_copy(st_mlp, out_mlp.at[row])
        return _

    lax.fori_loop(0, B_PER_W // 16, step, 0)


def _sc_gather(user, item, mfu_t, mfi_t, mlpu_t, mlpi_t):
    mesh = plsc.VectorSubcoreMesh(core_axis_name="c", subcore_axis_name="s")
    f32 = jnp.float32
    i32 = jnp.int32
    run = functools.partial(
        pl.kernel,
        mesh=mesh,
        compiler_params=pltpu.CompilerParams(needs_layout_passes=False),
        out_type=[
            jax.ShapeDtypeStruct((BATCH, MF_DIM), f32),
            jax.ShapeDtypeStruct((BATCH, MLP_DIM), f32),
        ],
        scratch_types=[
            pltpu.VMEM((B_PER_W,), i32),
            pltpu.VMEM((B_PER_W,), i32),
            pltpu.VMEM((SUB, MF_DIM, 128), f32),
            pltpu.VMEM((SUB, MF_DIM, 128), f32),
            pltpu.VMEM((SUB, MLP_HALF, 128), f32),
            pltpu.VMEM((SUB, MLP_HALF, 128), f32),
            pltpu.VMEM((16, MF_DIM), f32),
            pltpu.VMEM((16, MLP_DIM), f32),
            pltpu.SemaphoreType.DMA,
        ],
    )(_sc_gather_body)
    return run(user, item, mfu_t, mfi_t, mlpu_t, mlpi_t)


def _tc_mlp_body(xmf, xmlp, w0, b0, w1, b1, w2, b2, w3, b3, wfm, wfp, bf, out):
    f32 = jnp.float32
    h = xmlp[...]
    for w, b in ((w0, b0), (w1, b1), (w2, b2), (w3, b3)):
        h = jnp.maximum(jnp.dot(h, w[...], preferred_element_type=f32) + b[...], 0.0)
    logit = (jnp.dot(xmf[...], wfm[...], preferred_element_type=f32)
             + jnp.dot(h, wfp[...], preferred_element_type=f32)
             + bf[...])
    out[...] = jax.nn.sigmoid(logit)


def _tc_mlp(xmf, xmlp, W0, b0, W1, b1, W2, b2, W3, b3, Wf, bf):
    R = 2048
    grid = (BATCH // R,)
    D = MLP_DIM
    rows = lambda d: pl.BlockSpec((R, d), lambda i: (i, 0))
    full = lambda a, b: pl.BlockSpec((a, b), lambda i: (0, 0))
    in_specs = [
        rows(D), rows(D),
        full(D, D), full(1, D),
        full(D, D), full(1, D),
        full(D, D), full(1, D),
        full(D, D), full(1, D),
        full(D, 1), full(D, 1), full(1, 1),
    ]
    out_spec = pl.BlockSpec((R, 1), lambda i: (i, 0))
    args = (
        xmf, xmlp,
        W0, b0.reshape(1, D),
        W1, b1.reshape(1, D),
        W2, b2.reshape(1, D),
        W3, b3.reshape(1, D),
        Wf[:D], Wf[D:], bf.reshape(1, 1),
    )
    return pl.pallas_call(
        _tc_mlp_body,
        grid=grid,
        in_specs=in_specs,
        out_specs=out_spec,
        out_shape=jax.ShapeDtypeStruct((BATCH, 1), jnp.float32),
    )(*args)


def kernel(user, item, mf_user_embed, mf_item_embed, mlp_user_embed,
           mlp_item_embed, W0, b0, W1, b1, W2, b2, W3, b3, Wf, bf):
    xmf, xmlp = _sc_gather(user, item, mf_user_embed.T, mf_item_embed.T,
                           mlp_user_embed.T, mlp_item_embed.T)
    return _tc_mlp(xmf, xmlp, W0, b0, W1, b1, W2, b2, W3, b3, Wf, bf)
